# Initial kernel scaffold; baseline (speedup 1.0000x reference)
import jax
import jax.numpy as jnp
from jax.experimental import pallas as pl


def kernel(vision_input, text_input, W_text, b_text, W_ip, b_ip, bn_gamma, bn_beta, W_moe, b_moe, W_patch, b_patch, W_t2v, b_t2v, W1, b1, W2, b2, W_head, b_head):
    raise NotImplementedError("write your pallas kernel here")


# speedup vs baseline: 2.2850x; 2.2850x over previous
import functools

import jax
import jax.numpy as jnp
from jax import lax
from jax.experimental import pallas as pl
from jax.experimental.pallas import tpu as pltpu
from jax.experimental.pallas import tpu_sc as plsc

B = 4
HW = 384
P = 16
T = (HW // P) ** 2
D = 768
DI = 384
DFF = 3072
E = 8
TOPK = 2
NC = 21
LN2 = 0.6931471805599453

FBLK = 1536
NF = DFF // FBLK


def _gelu(x):
    return 0.5 * x * (1.0 + lax.erf(x * (2.0 ** -0.5)))


def _prep_kernel(text_ref, Wt_ref, bt_ref, Wm_ref, bm_ref, noise_ref,
                 Wip_ref, bip_ref, gam_ref, bet_ref, Wt2v_ref, bt2v_ref,
                 logits_ref, t2v_ref):
    tf = jnp.tanh(
        jnp.dot(text_ref[...], Wt_ref[...], preferred_element_type=jnp.float32)
        + bt_ref[...])
    logits = (jnp.dot(tf, Wm_ref[...], preferred_element_type=jnp.float32)
              + bm_ref[...] + noise_ref[...])
    pad = jnp.full((B, 16 - E), -1e30, jnp.float32)
    logits_ref[...] = jnp.concatenate([logits, pad], axis=1)

    h = (jnp.dot(tf, Wip_ref[...], preferred_element_type=jnp.float32)
         + bip_ref[...])
    mu = jnp.mean(h, axis=0, keepdims=True)
    var = jnp.mean((h - mu) ** 2, axis=0, keepdims=True)
    hn = (h - mu) / jnp.sqrt(var + 1e-5) * gam_ref[...] + bet_ref[...]
    instr = _gelu(hn)
    t2v_ref[...] = (jnp.dot(instr, Wt2v_ref[...],
                            preferred_element_type=jnp.float32)
                    + bt2v_ref[...])


def _embed_kernel(patches_ref, Wp_ref, bp_ref, t2v_ref, tok_ref):
    tok = (jnp.dot(patches_ref[0].astype(jnp.bfloat16),
                   Wp_ref[...].astype(jnp.bfloat16),
                   preferred_element_type=jnp.float32)
           + bp_ref[...] + t2v_ref[0])
    tok_ref[0] = tok.astype(jnp.bfloat16)


def _sc_log(x):
    bits = lax.bitcast_convert_type(x, jnp.int32)
    e = lax.shift_right_logical(bits, 23) - 127
    mbits = lax.bitwise_or(lax.bitwise_and(bits, 0x7FFFFF), 127 << 23)
    m = lax.bitcast_convert_type(mbits, jnp.float32)
    t = (m - 1.0) / (m + 1.0)
    t2 = t * t
    poly = 1.0 + t2 * (1.0 / 3.0 + t2 * (1.0 / 5.0 + t2 * (1.0 / 7.0
                                                           + t2 * (1.0 / 9.0))))
    return e.astype(jnp.float32) * LN2 + 2.0 * t * poly


def _sc_route_kernel(logits_hbm, scores_hbm, idx_hbm, misc_hbm,
                     lg_v, sc_v, idx_v, misc_v):
    c = lax.axis_index("c")
    s = lax.axis_index("s")

    @pl.when(jnp.logical_and(c == 0, s == 0))
    def _():
        pltpu.sync_copy(logits_hbm, lg_v)
        lanes = lax.iota(jnp.int32, 16)
        valid = lanes < E
        zero = jnp.zeros((16,), jnp.float32)
        bcast = lambda s: zero + s

        ssum = zero
        idxvec = jnp.zeros((16,), jnp.int32)
        ent_acc = zero
        for b in range(B):
            row = lg_v[b]
            mx = bcast(jnp.max(row, axis=0))
            p = jnp.exp(row - mx)
            p = p / bcast(jnp.sum(p, axis=0))
            scores = p * (1.0 / TOPK)
            sc_v[b] = scores
            ssum = ssum + scores
            ent_acc = ent_acc - bcast(jnp.sum(
                jnp.where(valid, scores * _sc_log(scores + 1e-8), zero),
                axis=0))

            m1 = bcast(jnp.max(p, axis=0))
            i1 = plsc.all_reduce_ffs(p == m1)
            p2 = jnp.where(lanes == i1, -jnp.inf, p)
            m2 = bcast(jnp.max(p2, axis=0))
            i2 = plsc.all_reduce_ffs(p2 == m2)
            idxvec = jnp.where(lanes == b, i1, idxvec)
            idxvec = jnp.where(lanes == (B + b), i2, idxvec)

        idx_v[...] = idxvec
        smean = bcast(jnp.sum(ssum, axis=0)) * (1.0 / E)
        dev = jnp.where(valid, ssum - smean, zero)
        svar = bcast(jnp.sum(dev * dev, axis=0)) * (1.0 / (E - 1))
        imp = svar / (smean * smean)
        ent = ent_acc * (1.0 / B)
        misc_v[...] = jnp.where(lanes == 0, imp,
                                jnp.where(lanes == 1, ent, zero))

        pltpu.sync_copy(sc_v, scores_hbm)
        pltpu.sync_copy(idx_v, idx_hbm)
        pltpu.sync_copy(misc_v, misc_hbm)


def _sc_route(logits_pad):
    mesh = plsc.VectorSubcoreMesh(core_axis_name="c", subcore_axis_name="s")
    f = pl.kernel(
        _sc_route_kernel,
        compiler_params=pltpu.CompilerParams(needs_layout_passes=False),
        out_type=(
            jax.ShapeDtypeStruct((B, 16), jnp.float32),
            jax.ShapeDtypeStruct((16,), jnp.int32),
            jax.ShapeDtypeStruct((16,), jnp.float32),
        ),
        mesh=mesh,
        scratch_types=[
            pltpu.VMEM((B, 16), jnp.float32),
            pltpu.VMEM((B, 16), jnp.float32),
            pltpu.VMEM((16,), jnp.int32),
            pltpu.VMEM((16,), jnp.float32),
        ],
    )
    return f(logits_pad)


def _ffn_kernel(idx_ref, tok_ref,
                W1a_ref, W1b_ref, b1a_ref, b1b_ref,
                W2a_ref, W2b_ref, b2a_ref, b2b_ref,
                Wh_ref, bh_ref, out_ref, acc_s):
    f = pl.program_id(1)

    @pl.when(f == 0)
    def _():
        acc_s[...] = jnp.zeros_like(acc_s)

    W1blk = (W1a_ref[0] + W1b_ref[0]).astype(jnp.bfloat16)
    b1blk = b1a_ref[0] + b1b_ref[0]
    hh = _gelu(jnp.dot(tok_ref[0], W1blk,
                       preferred_element_type=jnp.float32) + b1blk)
    W2blk = (W2a_ref[0] + W2b_ref[0]).astype(jnp.bfloat16)
    acc_s[...] += jnp.dot(hh.astype(jnp.bfloat16), W2blk,
                          preferred_element_type=jnp.float32)

    @pl.when(f == NF - 1)
    def _():
        outt = acc_s[...] + b2a_ref[0] + b2b_ref[0]
        out_ref[0] = (jnp.dot(outt, Wh_ref[...],
                              preferred_element_type=jnp.float32)
                      + bh_ref[...])


def kernel(vision_input, text_input, W_text, b_text, W_ip, b_ip, bn_gamma,
           bn_beta, W_moe, b_moe, W_patch, b_patch, W_t2v, b_t2v, W1, b1, W2,
           b2, W_head, b_head):
    f32 = jnp.float32
    bf16 = jnp.bfloat16
    noise = jax.random.normal(jax.random.key(42), (B, E), f32) / (E ** 2)

    row = lambda v: v.reshape(1, -1)

    logits_pad, t2v = pl.pallas_call(
        _prep_kernel,
        out_shape=(
            jax.ShapeDtypeStruct((B, 16), f32),
            jax.ShapeDtypeStruct((B, D), f32),
        ),
    )(text_input, W_text, row(b_text), W_moe, row(b_moe), noise,
      W_ip, row(b_ip), row(bn_gamma), row(bn_beta), W_t2v, row(b_t2v))

    x = vision_input.reshape(B, 3, HW // P, P, HW // P, P)
    patches = x.transpose(0, 2, 4, 1, 3, 5).reshape(B, T, 3 * P * P)

    tokens = pl.pallas_call(
        _embed_kernel,
        grid=(B,),
        in_specs=[
            pl.BlockSpec((1, T, D), lambda b: (b, 0, 0)),
            pl.BlockSpec((D, D), lambda b: (0, 0)),
            pl.BlockSpec((1, D), lambda b: (0, 0)),
            pl.BlockSpec((1, 1, D), lambda b: (b, 0, 0)),
        ],
        out_specs=pl.BlockSpec((1, T, D), lambda b: (b, 0, 0)),
        out_shape=jax.ShapeDtypeStruct((B, T, D), bf16),
    )(patches, W_patch, row(b_patch), t2v.reshape(B, 1, D))

    scores16, idx16, misc16 = _sc_route(logits_pad)

    grid_spec = pltpu.PrefetchScalarGridSpec(
        num_scalar_prefetch=1,
        grid=(B, NF),
        in_specs=[
            pl.BlockSpec((1, T, D), lambda b, f, idx: (b, 0, 0)),
            pl.BlockSpec((1, D, FBLK), lambda b, f, idx: (idx[b], 0, f)),
            pl.BlockSpec((1, D, FBLK), lambda b, f, idx: (idx[B + b], 0, f)),
            pl.BlockSpec((1, 1, FBLK), lambda b, f, idx: (idx[b], 0, f)),
            pl.BlockSpec((1, 1, FBLK), lambda b, f, idx: (idx[B + b], 0, f)),
            pl.BlockSpec((1, FBLK, D), lambda b, f, idx: (idx[b], f, 0)),
            pl.BlockSpec((1, FBLK, D), lambda b, f, idx: (idx[B + b], f, 0)),
            pl.BlockSpec((1, 1, D), lambda b, f, idx: (idx[b], 0, 0)),
            pl.BlockSpec((1, 1, D), lambda b, f, idx: (idx[B + b], 0, 0)),
            pl.BlockSpec((D, NC), lambda b, f, idx: (0, 0)),
            pl.BlockSpec((1, NC), lambda b, f, idx: (0, 0)),
        ],
        out_specs=pl.BlockSpec((1, T, NC), lambda b, f, idx: (b, 0, 0)),
        scratch_shapes=[
            pltpu.VMEM((T, D), f32),
        ],
    )

    logits = pl.pallas_call(
        _ffn_kernel,
        grid_spec=grid_spec,
        out_shape=jax.ShapeDtypeStruct((B, T, NC), f32),
        compiler_params=pltpu.CompilerParams(
            dimension_semantics=("arbitrary", "arbitrary"),
        ),
    )(idx16, tokens,
      W1, W1,
      b1.reshape(E, 1, DFF), b1.reshape(E, 1, DFF),
      W2, W2,
      b2.reshape(E, 1, D), b2.reshape(E, 1, D),
      W_head, row(b_head))

    return (logits, scores16[:, :E], misc16[0].reshape(()),
            misc16[1].reshape(()))


# speedup vs baseline: 3.0829x; 1.3492x over previous
import functools

import jax
import jax.numpy as jnp
from jax import lax
from jax.experimental import pallas as pl
from jax.experimental.pallas import tpu as pltpu
from jax.experimental.pallas import tpu_sc as plsc

B = 4
HW = 384
P = 16
T = (HW // P) ** 2
D = 768
DI = 384
DFF = 3072
E = 8
TOPK = 2
NC = 21
LN2 = 0.6931471805599453

FBLK = 1024
NF = DFF // FBLK


def _gelu(x):
    return 0.5 * x * (1.0 + lax.erf(x * (2.0 ** -0.5)))


def _prep_kernel(text_ref, Wt_ref, bt_ref, Wm_ref, bm_ref, noise_ref,
                 Wip_ref, bip_ref, gam_ref, bet_ref, Wt2v_ref, bt2v_ref,
                 logits_ref, t2v_ref):
    tf = jnp.tanh(
        jnp.dot(text_ref[...], Wt_ref[...], preferred_element_type=jnp.float32)
        + bt_ref[...])
    logits = (jnp.dot(tf, Wm_ref[...], preferred_element_type=jnp.float32)
              + bm_ref[...] + noise_ref[...])
    pad = jnp.full((B, 16 - E), -1e30, jnp.float32)
    logits_ref[...] = jnp.concatenate([logits, pad], axis=1)

    h = (jnp.dot(tf, Wip_ref[...], preferred_element_type=jnp.float32)
         + bip_ref[...])
    mu = jnp.mean(h, axis=0, keepdims=True)
    var = jnp.mean((h - mu) ** 2, axis=0, keepdims=True)
    hn = (h - mu) / jnp.sqrt(var + 1e-5) * gam_ref[...] + bet_ref[...]
    instr = _gelu(hn)
    t2v_ref[...] = (jnp.dot(instr, Wt2v_ref[...],
                            preferred_element_type=jnp.float32)
                    + bt2v_ref[...])


def _sc_log(x):
    bits = lax.bitcast_convert_type(x, jnp.int32)
    e = lax.shift_right_logical(bits, 23) - 127
    mbits = lax.bitwise_or(lax.bitwise_and(bits, 0x7FFFFF), 127 << 23)
    m = lax.bitcast_convert_type(mbits, jnp.float32)
    t = (m - 1.0) / (m + 1.0)
    t2 = t * t
    poly = 1.0 + t2 * (1.0 / 3.0 + t2 * (1.0 / 5.0 + t2 * (1.0 / 7.0
                                                           + t2 * (1.0 / 9.0))))
    return e.astype(jnp.float32) * LN2 + 2.0 * t * poly


def _sc_route_kernel(logits_hbm, scores_hbm, idx_hbm, misc_hbm,
                     lg_v, sc_v, idx_v, misc_v):
    c = lax.axis_index("c")
    s = lax.axis_index("s")

    @pl.when(jnp.logical_and(c == 0, s == 0))
    def _():
        pltpu.sync_copy(logits_hbm, lg_v)
        lanes = lax.iota(jnp.int32, 16)
        valid = lanes < E
        zero = jnp.zeros((16,), jnp.float32)
        bcast = lambda s: zero + s

        ssum = zero
        idxvec = jnp.zeros((16,), jnp.int32)
        ent_acc = zero
        for b in range(B):
            row = lg_v[b]
            mx = bcast(jnp.max(row, axis=0))
            p = jnp.exp(row - mx)
            p = p / bcast(jnp.sum(p, axis=0))
            scores = p * (1.0 / TOPK)
            sc_v[b] = scores
            ssum = ssum + scores
            ent_acc = ent_acc - bcast(jnp.sum(
                jnp.where(valid, scores * _sc_log(scores + 1e-8), zero),
                axis=0))

            m1 = bcast(jnp.max(p, axis=0))
            i1 = plsc.all_reduce_ffs(p == m1)
            p2 = jnp.where(lanes == i1, -jnp.inf, p)
            m2 = bcast(jnp.max(p2, axis=0))
            i2 = plsc.all_reduce_ffs(p2 == m2)
            idxvec = jnp.where(lanes == b, i1, idxvec)
            idxvec = jnp.where(lanes == (B + b), i2, idxvec)

        idx_v[...] = idxvec
        smean = bcast(jnp.sum(ssum, axis=0)) * (1.0 / E)
        dev = jnp.where(valid, ssum - smean, zero)
        svar = bcast(jnp.sum(dev * dev, axis=0)) * (1.0 / (E - 1))
        imp = svar / (smean * smean)
        ent = ent_acc * (1.0 / B)
        misc_v[...] = jnp.where(lanes == 0, imp,
                                jnp.where(lanes == 1, ent, zero))

        pltpu.sync_copy(sc_v, scores_hbm)
        pltpu.sync_copy(idx_v, idx_hbm)
        pltpu.sync_copy(misc_v, misc_hbm)


def _sc_route(logits_pad):
    mesh = plsc.VectorSubcoreMesh(core_axis_name="c", subcore_axis_name="s")
    f = pl.kernel(
        _sc_route_kernel,
        compiler_params=pltpu.CompilerParams(needs_layout_passes=False),
        out_type=(
            jax.ShapeDtypeStruct((B, 16), jnp.float32),
            jax.ShapeDtypeStruct((16,), jnp.int32),
            jax.ShapeDtypeStruct((16,), jnp.float32),
        ),
        mesh=mesh,
        scratch_types=[
            pltpu.VMEM((B, 16), jnp.float32),
            pltpu.VMEM((B, 16), jnp.float32),
            pltpu.VMEM((16,), jnp.int32),
            pltpu.VMEM((16,), jnp.float32),
        ],
    )
    return f(logits_pad)


def _ffn_kernel(idx_ref, xr_ref, Wp_ref, bp_ref, t2v_ref,
                W1a_ref, W1b_ref, b1a_ref, b1b_ref,
                W2a_ref, W2b_ref, b2a_ref, b2b_ref,
                Wh_ref, bh_ref, out_ref, tok_s, acc_s):
    f = pl.program_id(1)

    @pl.when(f == 0)
    def _():
        cols = []
        for c in range(3):
            v = xr_ref[0, c]
            for py in range(P):
                cols.append(v[:, py].reshape(T, P))
        patches = jnp.concatenate(cols, axis=1)
        tok = (jnp.dot(patches, Wp_ref[...].astype(jnp.bfloat16),
                       preferred_element_type=jnp.float32)
               + bp_ref[...] + t2v_ref[0])
        tok_s[...] = tok.astype(jnp.bfloat16)
        acc_s[...] = jnp.zeros_like(acc_s)

    W1blk = (W1a_ref[0] + W1b_ref[0]).astype(jnp.bfloat16)
    b1blk = b1a_ref[0] + b1b_ref[0]
    hh = _gelu(jnp.dot(tok_s[...], W1blk,
                       preferred_element_type=jnp.float32) + b1blk)
    W2blk = (W2a_ref[0] + W2b_ref[0]).astype(jnp.bfloat16)
    acc_s[...] += jnp.dot(hh.astype(jnp.bfloat16), W2blk,
                          preferred_element_type=jnp.float32)

    @pl.when(f == NF - 1)
    def _():
        outt = acc_s[...] + b2a_ref[0] + b2b_ref[0]
        out_ref[0] = (jnp.dot(outt, Wh_ref[...],
                              preferred_element_type=jnp.float32)
                      + bh_ref[...])


def kernel(vision_input, text_input, W_text, b_text, W_ip, b_ip, bn_gamma,
           bn_beta, W_moe, b_moe, W_patch, b_patch, W_t2v, b_t2v, W1, b1, W2,
           b2, W_head, b_head):
    f32 = jnp.float32
    bf16 = jnp.bfloat16
    noise = jax.random.normal(jax.random.key(42), (B, E), f32) / (E ** 2)

    row = lambda v: v.reshape(1, -1)

    logits_pad, t2v = pl.pallas_call(
        _prep_kernel,
        out_shape=(
            jax.ShapeDtypeStruct((B, 16), f32),
            jax.ShapeDtypeStruct((B, D), f32),
        ),
    )(text_input, W_text, row(b_text), W_moe, row(b_moe), noise,
      W_ip, row(b_ip), row(bn_gamma), row(bn_beta), W_t2v, row(b_t2v))

    xr = vision_input.astype(bf16).reshape(B, 3, HW // P, P, HW // P, P)

    scores16, idx16, misc16 = _sc_route(logits_pad)

    grid_spec = pltpu.PrefetchScalarGridSpec(
        num_scalar_prefetch=1,
        grid=(B, NF),
        in_specs=[
            pl.BlockSpec((1, 3, HW // P, P, HW // P, P),
                         lambda b, f, idx: (b, 0, 0, 0, 0, 0)),
            pl.BlockSpec((D, D), lambda b, f, idx: (0, 0)),
            pl.BlockSpec((1, D), lambda b, f, idx: (0, 0)),
            pl.BlockSpec((1, 1, D), lambda b, f, idx: (b, 0, 0)),
            pl.BlockSpec((1, D, FBLK), lambda b, f, idx: (idx[b], 0, f)),
            pl.BlockSpec((1, D, FBLK), lambda b, f, idx: (idx[B + b], 0, f)),
            pl.BlockSpec((1, 1, FBLK), lambda b, f, idx: (idx[b], 0, f)),
            pl.BlockSpec((1, 1, FBLK), lambda b, f, idx: (idx[B + b], 0, f)),
            pl.BlockSpec((1, FBLK, D), lambda b, f, idx: (idx[b], f, 0)),
            pl.BlockSpec((1, FBLK, D), lambda b, f, idx: (idx[B + b], f, 0)),
            pl.BlockSpec((1, 1, D), lambda b, f, idx: (idx[b], 0, 0)),
            pl.BlockSpec((1, 1, D), lambda b, f, idx: (idx[B + b], 0, 0)),
            pl.BlockSpec((D, NC), lambda b, f, idx: (0, 0)),
            pl.BlockSpec((1, NC), lambda b, f, idx: (0, 0)),
        ],
        out_specs=pl.BlockSpec((1, T, NC), lambda b, f, idx: (b, 0, 0)),
        scratch_shapes=[
            pltpu.VMEM((T, D), bf16),
            pltpu.VMEM((T, D), f32),
        ],
    )

    logits = pl.pallas_call(
        _ffn_kernel,
        grid_spec=grid_spec,
        out_shape=jax.ShapeDtypeStruct((B, T, NC), f32),
        compiler_params=pltpu.CompilerParams(
            dimension_semantics=("arbitrary", "arbitrary"),
        ),
    )(idx16, xr, W_patch, row(b_patch), t2v.reshape(B, 1, D),
      W1, W1,
      b1.reshape(E, 1, DFF), b1.reshape(E, 1, DFF),
      W2, W2,
      b2.reshape(E, 1, D), b2.reshape(E, 1, D),
      W_head, row(b_head))

    return (logits, scores16[:, :E], misc16[0].reshape(()),
            misc16[1].reshape(()))


# speedup vs baseline: 3.1289x; 1.0149x over previous
import functools

import jax
import jax.numpy as jnp
from jax import lax
from jax.experimental import pallas as pl
from jax.experimental.pallas import tpu as pltpu
from jax.experimental.pallas import tpu_sc as plsc

B = 4
HW = 384
P = 16
T = (HW // P) ** 2
D = 768
DI = 384
DFF = 3072
E = 8
TOPK = 2
NC = 21
LN2 = 0.6931471805599453

FBLK = 1536
NF = DFF // FBLK


def _gelu(x):
    return 0.5 * x * (1.0 + lax.erf(x * (2.0 ** -0.5)))


def _prep_kernel(text_ref, Wt_ref, bt_ref, Wm_ref, bm_ref, noise_ref,
                 Wip_ref, bip_ref, gam_ref, bet_ref, Wt2v_ref, bt2v_ref,
                 logits_ref, t2v_ref):
    tf = jnp.tanh(
        jnp.dot(text_ref[...], Wt_ref[...], preferred_element_type=jnp.float32)
        + bt_ref[...])
    logits = (jnp.dot(tf, Wm_ref[...], preferred_element_type=jnp.float32)
              + bm_ref[...] + noise_ref[...])
    pad = jnp.full((B, 16 - E), -1e30, jnp.float32)
    logits_ref[...] = jnp.concatenate([logits, pad], axis=1)

    h = (jnp.dot(tf, Wip_ref[...], preferred_element_type=jnp.float32)
         + bip_ref[...])
    mu = jnp.mean(h, axis=0, keepdims=True)
    var = jnp.mean((h - mu) ** 2, axis=0, keepdims=True)
    hn = (h - mu) / jnp.sqrt(var + 1e-5) * gam_ref[...] + bet_ref[...]
    instr = _gelu(hn)
    t2v_ref[...] = (jnp.dot(instr, Wt2v_ref[...],
                            preferred_element_type=jnp.float32)
                    + bt2v_ref[...])


def _embed_kernel(xr_ref, Wp_ref, bp_ref, t2v_ref, tok_ref):
    cols = []
    for c in range(3):
        v = xr_ref[0, c]
        for py in range(P):
            cols.append(v[:, py].reshape(T, P))
    patches = jnp.concatenate(cols, axis=1)
    tok = (jnp.dot(patches, Wp_ref[...].astype(jnp.bfloat16),
                   preferred_element_type=jnp.float32)
           + bp_ref[...] + t2v_ref[0])
    tok_ref[0] = tok.astype(jnp.bfloat16)


def _sc_log(x):
    bits = lax.bitcast_convert_type(x, jnp.int32)
    e = lax.shift_right_logical(bits, 23) - 127
    mbits = lax.bitwise_or(lax.bitwise_and(bits, 0x7FFFFF), 127 << 23)
    m = lax.bitcast_convert_type(mbits, jnp.float32)
    t = (m - 1.0) / (m + 1.0)
    t2 = t * t
    poly = 1.0 + t2 * (1.0 / 3.0 + t2 * (1.0 / 5.0 + t2 * (1.0 / 7.0
                                                           + t2 * (1.0 / 9.0))))
    return e.astype(jnp.float32) * LN2 + 2.0 * t * poly


def _sc_route_kernel(logits_hbm, scores_hbm, idx_hbm, misc_hbm,
                     lg_v, sc_v, idx_v, misc_v):
    c = lax.axis_index("c")
    s = lax.axis_index("s")

    @pl.when(jnp.logical_and(c == 0, s == 0))
    def _():
        pltpu.sync_copy(logits_hbm, lg_v)
        lanes = lax.iota(jnp.int32, 16)
        valid = lanes < E
        zero = jnp.zeros((16,), jnp.float32)
        bcast = lambda s: zero + s

        ssum = zero
        idxvec = jnp.zeros((16,), jnp.int32)
        ent_acc = zero
        for b in range(B):
            row = lg_v[b]
            mx = bcast(jnp.max(row, axis=0))
            p = jnp.exp(row - mx)
            p = p / bcast(jnp.sum(p, axis=0))
            scores = p * (1.0 / TOPK)
            sc_v[b] = scores
            ssum = ssum + scores
            ent_acc = ent_acc - bcast(jnp.sum(
                jnp.where(valid, scores * _sc_log(scores + 1e-8), zero),
                axis=0))

            m1 = bcast(jnp.max(p, axis=0))
            i1 = plsc.all_reduce_ffs(p == m1)
            p2 = jnp.where(lanes == i1, -jnp.inf, p)
            m2 = bcast(jnp.max(p2, axis=0))
            i2 = plsc.all_reduce_ffs(p2 == m2)
            idxvec = jnp.where(lanes == b, i1, idxvec)
            idxvec = jnp.where(lanes == (B + b), i2, idxvec)

        idx_v[...] = idxvec
        smean = bcast(jnp.sum(ssum, axis=0)) * (1.0 / E)
        dev = jnp.where(valid, ssum - smean, zero)
        svar = bcast(jnp.sum(dev * dev, axis=0)) * (1.0 / (E - 1))
        imp = svar / (smean * smean)
        ent = ent_acc * (1.0 / B)
        misc_v[...] = jnp.where(lanes == 0, imp,
                                jnp.where(lanes == 1, ent, zero))

        pltpu.sync_copy(sc_v, scores_hbm)
        pltpu.sync_copy(idx_v, idx_hbm)
        pltpu.sync_copy(misc_v, misc_hbm)


def _sc_route(logits_pad):
    mesh = plsc.VectorSubcoreMesh(core_axis_name="c", subcore_axis_name="s")
    f = pl.kernel(
        _sc_route_kernel,
        compiler_params=pltpu.CompilerParams(needs_layout_passes=False),
        out_type=(
            jax.ShapeDtypeStruct((B, 16), jnp.float32),
            jax.ShapeDtypeStruct((16,), jnp.int32),
            jax.ShapeDtypeStruct((16,), jnp.float32),
        ),
        mesh=mesh,
        scratch_types=[
            pltpu.VMEM((B, 16), jnp.float32),
            pltpu.VMEM((B, 16), jnp.float32),
            pltpu.VMEM((16,), jnp.int32),
            pltpu.VMEM((16,), jnp.float32),
        ],
    )
    return f(logits_pad)


def _ffn_kernel(idx_ref, tok_ref,
                W1a_ref, W1b_ref, b1a_ref, b1b_ref,
                W2a_ref, W2b_ref, b2a_ref, b2b_ref,
                Wh_ref, bh_ref, out_ref, acc_s):
    f = pl.program_id(1)

    @pl.when(f == 0)
    def _():
        acc_s[...] = jnp.zeros_like(acc_s)

    W1blk = (W1a_ref[0] + W1b_ref[0]).astype(jnp.bfloat16)
    b1blk = b1a_ref[0] + b1b_ref[0]
    hh = _gelu(jnp.dot(tok_ref[0], W1blk,
                       preferred_element_type=jnp.float32) + b1blk)
    W2blk = (W2a_ref[0] + W2b_ref[0]).astype(jnp.bfloat16)
    acc_s[...] += jnp.dot(hh.astype(jnp.bfloat16), W2blk,
                          preferred_element_type=jnp.float32)

    @pl.when(f == NF - 1)
    def _():
        outt = acc_s[...] + b2a_ref[0] + b2b_ref[0]
        out_ref[0] = (jnp.dot(outt, Wh_ref[...],
                              preferred_element_type=jnp.float32)
                      + bh_ref[...])


def kernel(vision_input, text_input, W_text, b_text, W_ip, b_ip, bn_gamma,
           bn_beta, W_moe, b_moe, W_patch, b_patch, W_t2v, b_t2v, W1, b1, W2,
           b2, W_head, b_head):
    f32 = jnp.float32
    bf16 = jnp.bfloat16
    noise = jax.random.normal(jax.random.key(42), (B, E), f32) / (E ** 2)

    row = lambda v: v.reshape(1, -1)

    logits_pad, t2v = pl.pallas_call(
        _prep_kernel,
        out_shape=(
            jax.ShapeDtypeStruct((B, 16), f32),
            jax.ShapeDtypeStruct((B, D), f32),
        ),
    )(text_input, W_text, row(b_text), W_moe, row(b_moe), noise,
      W_ip, row(b_ip), row(bn_gamma), row(bn_beta), W_t2v, row(b_t2v))

    xr = vision_input.astype(bf16).reshape(B, 3, HW // P, P, HW // P, P)

    tokens = pl.pallas_call(
        _embed_kernel,
        grid=(B,),
        in_specs=[
            pl.BlockSpec((1, 3, HW // P, P, HW // P, P),
                         lambda b: (b, 0, 0, 0, 0, 0)),
            pl.BlockSpec((D, D), lambda b: (0, 0)),
            pl.BlockSpec((1, D), lambda b: (0, 0)),
            pl.BlockSpec((1, 1, D), lambda b: (b, 0, 0)),
        ],
        out_specs=pl.BlockSpec((1, T, D), lambda b: (b, 0, 0)),
        out_shape=jax.ShapeDtypeStruct((B, T, D), bf16),
    )(xr, W_patch, row(b_patch), t2v.reshape(B, 1, D))

    scores16, idx16, misc16 = _sc_route(logits_pad)

    grid_spec = pltpu.PrefetchScalarGridSpec(
        num_scalar_prefetch=1,
        grid=(B, NF),
        in_specs=[
            pl.BlockSpec((1, T, D), lambda b, f, idx: (b, 0, 0)),
            pl.BlockSpec((1, D, FBLK), lambda b, f, idx: (idx[b], 0, f)),
            pl.BlockSpec((1, D, FBLK), lambda b, f, idx: (idx[B + b], 0, f)),
            pl.BlockSpec((1, 1, FBLK), lambda b, f, idx: (idx[b], 0, f)),
            pl.BlockSpec((1, 1, FBLK), lambda b, f, idx: (idx[B + b], 0, f)),
            pl.BlockSpec((1, FBLK, D), lambda b, f, idx: (idx[b], f, 0)),
            pl.BlockSpec((1, FBLK, D), lambda b, f, idx: (idx[B + b], f, 0)),
            pl.BlockSpec((1, 1, D), lambda b, f, idx: (idx[b], 0, 0)),
            pl.BlockSpec((1, 1, D), lambda b, f, idx: (idx[B + b], 0, 0)),
            pl.BlockSpec((D, NC), lambda b, f, idx: (0, 0)),
            pl.BlockSpec((1, NC), lambda b, f, idx: (0, 0)),
        ],
        out_specs=pl.BlockSpec((1, T, NC), lambda b, f, idx: (b, 0, 0)),
        scratch_shapes=[
            pltpu.VMEM((T, D), f32),
        ],
    )

    logits = pl.pallas_call(
        _ffn_kernel,
        grid_spec=grid_spec,
        out_shape=jax.ShapeDtypeStruct((B, T, NC), f32),
        compiler_params=pltpu.CompilerParams(
            dimension_semantics=("arbitrary", "arbitrary"),
        ),
    )(idx16, tokens,
      W1, W1,
      b1.reshape(E, 1, DFF), b1.reshape(E, 1, DFF),
      W2, W2,
      b2.reshape(E, 1, D), b2.reshape(E, 1, D),
      W_head, row(b_head))

    return (logits, scores16[:, :E], misc16[0].reshape(()),
            misc16[1].reshape(()))


# speedup vs baseline: 3.1381x; 1.0029x over previous
import jax
import jax.numpy as jnp
from jax import lax
from jax.experimental import pallas as pl
from jax.experimental.pallas import tpu as pltpu
from jax.experimental.pallas import tpu_sc as plsc

B = 4
HW = 384
P = 16
T = (HW // P) ** 2
D = 768
DI = 384
DFF = 3072
E = 8
TOPK = 2
NC = 21
LN2 = 0.6931471805599453

FBLK = 1536
NF = DFF // FBLK


def _gelu(x):
    return 0.5 * x * (1.0 + lax.erf(x * (2.0 ** -0.5)))


def _prep_kernel(text_ref, Wt_ref, bt_ref, Wm_ref, bm_ref, noise_ref,
                 Wip_ref, bip_ref, gam_ref, bet_ref, Wt2v_ref, bt2v_ref,
                 logits_ref, t2v_ref):
    tf = jnp.tanh(
        jnp.dot(text_ref[...], Wt_ref[...], preferred_element_type=jnp.float32)
        + bt_ref[...])
    logits = (jnp.dot(tf, Wm_ref[...], preferred_element_type=jnp.float32)
              + bm_ref[...] + noise_ref[...])
    pad = jnp.full((B, 16 - E), -1e30, jnp.float32)
    logits_ref[...] = jnp.concatenate([logits, pad], axis=1)

    h = (jnp.dot(tf, Wip_ref[...], preferred_element_type=jnp.float32)
         + bip_ref[...])
    mu = jnp.mean(h, axis=0, keepdims=True)
    var = jnp.mean((h - mu) ** 2, axis=0, keepdims=True)
    hn = (h - mu) / jnp.sqrt(var + 1e-5) * gam_ref[...] + bet_ref[...]
    instr = _gelu(hn)
    t2v_ref[...] = (jnp.dot(instr, Wt2v_ref[...],
                            preferred_element_type=jnp.float32)
                    + bt2v_ref[...])


def _embed_kernel(xr_ref, Wp_ref, bp_ref, t2v_ref, tok_ref):
    cols = []
    for c in range(3):
        v = xr_ref[0, c]
        for py in range(P):
            cols.append(v[:, py].reshape(T, P))
    patches = jnp.concatenate(cols, axis=1)
    tok = (jnp.dot(patches, Wp_ref[...].astype(jnp.bfloat16),
                   preferred_element_type=jnp.float32)
           + bp_ref[...] + t2v_ref[0])
    tok_ref[0] = tok.astype(jnp.bfloat16)


def _sc_log(x):
    bits = lax.bitcast_convert_type(x, jnp.int32)
    e = lax.shift_right_logical(bits, 23) - 127
    mbits = lax.bitwise_or(lax.bitwise_and(bits, 0x7FFFFF), 127 << 23)
    m = lax.bitcast_convert_type(mbits, jnp.float32)
    t = (m - 1.0) / (m + 1.0)
    t2 = t * t
    poly = 1.0 + t2 * (1.0 / 3.0 + t2 * (1.0 / 5.0 + t2 * (1.0 / 7.0
                                                           + t2 * (1.0 / 9.0))))
    return e.astype(jnp.float32) * LN2 + 2.0 * t * poly


def _sc_route_kernel(logits_hbm, scores_hbm, idx_hbm, misc_hbm,
                     lg_v, sc_v, idx_v, misc_v):
    c = lax.axis_index("c")
    s = lax.axis_index("s")

    @pl.when(jnp.logical_and(c == 0, s == 0))
    def _():
        pltpu.sync_copy(logits_hbm, lg_v)
        lanes = lax.iota(jnp.int32, 16)
        valid = lanes < E
        zero = jnp.zeros((16,), jnp.float32)
        bcast = lambda s: zero + s

        ssum = zero
        idxvec = jnp.zeros((16,), jnp.int32)
        ent_acc = zero
        for b in range(B):
            row = lg_v[b]
            mx = bcast(jnp.max(row, axis=0))
            p = jnp.exp(row - mx)
            p = p / bcast(jnp.sum(p, axis=0))
            scores = p * (1.0 / TOPK)
            sc_v[b] = scores
            ssum = ssum + scores
            ent_acc = ent_acc - bcast(jnp.sum(
                jnp.where(valid, scores * _sc_log(scores + 1e-8), zero),
                axis=0))

            m1 = bcast(jnp.max(p, axis=0))
            i1 = plsc.all_reduce_ffs(p == m1)
            p2 = jnp.where(lanes == i1, -jnp.inf, p)
            m2 = bcast(jnp.max(p2, axis=0))
            i2 = plsc.all_reduce_ffs(p2 == m2)
            idxvec = jnp.where(lanes == b, i1, idxvec)
            idxvec = jnp.where(lanes == (B + b), i2, idxvec)

        idx_v[...] = idxvec
        smean = bcast(jnp.sum(ssum, axis=0)) * (1.0 / E)
        dev = jnp.where(valid, ssum - smean, zero)
        svar = bcast(jnp.sum(dev * dev, axis=0)) * (1.0 / (E - 1))
        imp = svar / (smean * smean)
        ent = ent_acc * (1.0 / B)
        misc_v[...] = jnp.where(lanes == 0, imp,
                                jnp.where(lanes == 1, ent, zero))

        pltpu.sync_copy(sc_v, scores_hbm)
        pltpu.sync_copy(idx_v, idx_hbm)
        pltpu.sync_copy(misc_v, misc_hbm)


def _sc_route(logits_pad):
    mesh = plsc.VectorSubcoreMesh(core_axis_name="c", subcore_axis_name="s")
    f = pl.kernel(
        _sc_route_kernel,
        compiler_params=pltpu.CompilerParams(needs_layout_passes=False),
        out_type=(
            jax.ShapeDtypeStruct((B, 16), jnp.float32),
            jax.ShapeDtypeStruct((16,), jnp.int32),
            jax.ShapeDtypeStruct((16,), jnp.float32),
        ),
        mesh=mesh,
        scratch_types=[
            pltpu.VMEM((B, 16), jnp.float32),
            pltpu.VMEM((B, 16), jnp.float32),
            pltpu.VMEM((16,), jnp.int32),
            pltpu.VMEM((16,), jnp.float32),
        ],
    )
    return f(logits_pad)


def _ffn_kernel(idx_ref, tok_ref,
                W1a_ref, W1b_ref, b1a_ref, b1b_ref,
                W2a_ref, W2b_ref, b2a_ref, b2b_ref,
                Wh_ref, bh_ref, out_ref, acc_s):
    f = pl.program_id(1)

    @pl.when(f == 0)
    def _():
        acc_s[...] = jnp.zeros_like(acc_s)

    W1blk = (W1a_ref[0] + W1b_ref[0]).astype(jnp.bfloat16)
    b1blk = b1a_ref[0] + b1b_ref[0]
    hh = _gelu(jnp.dot(tok_ref[0], W1blk,
                       preferred_element_type=jnp.float32) + b1blk)
    W2blk = (W2a_ref[0] + W2b_ref[0]).astype(jnp.bfloat16)
    acc_s[...] += jnp.dot(hh.astype(jnp.bfloat16), W2blk,
                          preferred_element_type=jnp.float32)

    @pl.when(f == NF - 1)
    def _():
        outt = acc_s[...] + b2a_ref[0] + b2b_ref[0]
        out_ref[0] = (jnp.dot(outt, Wh_ref[...],
                              preferred_element_type=jnp.float32)
                      + bh_ref[...])


def kernel(vision_input, text_input, W_text, b_text, W_ip, b_ip, bn_gamma,
           bn_beta, W_moe, b_moe, W_patch, b_patch, W_t2v, b_t2v, W1, b1, W2,
           b2, W_head, b_head):
    f32 = jnp.float32
    bf16 = jnp.bfloat16
    noise = jax.random.normal(jax.random.key(42), (B, E), f32) / (E ** 2)

    row = lambda v: v.reshape(1, -1)

    logits_pad, t2v = pl.pallas_call(
        _prep_kernel,
        out_shape=(
            jax.ShapeDtypeStruct((B, 16), f32),
            jax.ShapeDtypeStruct((B, D), f32),
        ),
    )(text_input, W_text, row(b_text), W_moe, row(b_moe), noise,
      W_ip, row(b_ip), row(bn_gamma), row(bn_beta), W_t2v, row(b_t2v))

    xr = vision_input.astype(bf16).reshape(B, 3, HW // P, P, HW // P, P)

    tokens = pl.pallas_call(
        _embed_kernel,
        grid=(B,),
        in_specs=[
            pl.BlockSpec((1, 3, HW // P, P, HW // P, P),
                         lambda b: (b, 0, 0, 0, 0, 0)),
            pl.BlockSpec((D, D), lambda b: (0, 0)),
            pl.BlockSpec((1, D), lambda b: (0, 0)),
            pl.BlockSpec((1, 1, D), lambda b: (b, 0, 0)),
        ],
        out_specs=pl.BlockSpec((1, T, D), lambda b: (b, 0, 0)),
        out_shape=jax.ShapeDtypeStruct((B, T, D), bf16),
    )(xr, W_patch, row(b_patch), t2v.reshape(B, 1, D))

    scores16, idx16, misc16 = _sc_route(logits_pad)

    grid_spec = pltpu.PrefetchScalarGridSpec(
        num_scalar_prefetch=1,
        grid=(B, NF),
        in_specs=[
            pl.BlockSpec((1, T, D), lambda b, f, idx: (b, 0, 0)),
            pl.BlockSpec((1, D, FBLK), lambda b, f, idx: (idx[b], 0, f)),
            pl.BlockSpec((1, D, FBLK), lambda b, f, idx: (idx[B + b], 0, f)),
            pl.BlockSpec((1, 1, FBLK), lambda b, f, idx: (idx[b], 0, f)),
            pl.BlockSpec((1, 1, FBLK), lambda b, f, idx: (idx[B + b], 0, f)),
            pl.BlockSpec((1, FBLK, D), lambda b, f, idx: (idx[b], f, 0)),
            pl.BlockSpec((1, FBLK, D), lambda b, f, idx: (idx[B + b], f, 0)),
            pl.BlockSpec((1, 1, D), lambda b, f, idx: (idx[b], 0, 0)),
            pl.BlockSpec((1, 1, D), lambda b, f, idx: (idx[B + b], 0, 0)),
            pl.BlockSpec((D, NC), lambda b, f, idx: (0, 0)),
            pl.BlockSpec((1, NC), lambda b, f, idx: (0, 0)),
        ],
        out_specs=pl.BlockSpec((1, T, NC), lambda b, f, idx: (b, 0, 0)),
        scratch_shapes=[
            pltpu.VMEM((T, D), f32),
        ],
    )

    logits = pl.pallas_call(
        _ffn_kernel,
        grid_spec=grid_spec,
        out_shape=jax.ShapeDtypeStruct((B, T, NC), f32),
        compiler_params=pltpu.CompilerParams(
            dimension_semantics=("arbitrary", "arbitrary"),
        ),
    )(idx16, tokens,
      W1, W1,
      b1.reshape(E, 1, DFF), b1.reshape(E, 1, DFF),
      W2, W2,
      b2.reshape(E, 1, D), b2.reshape(E, 1, D),
      W_head, row(b_head))

    return (logits, scores16[:, :E], misc16[0].reshape(()),
            misc16[1].reshape(()))
